# Initial kernel scaffold; baseline (speedup 1.0000x reference)
import functools

import jax
import jax.numpy as jnp
from jax import lax
from jax.experimental import pallas as pl
from jax.experimental.pallas import tpu as pltpu
from jax.experimental.pallas import tpu_sc as plsc

DIM = 64
IDXW = 128
CHUNK = 512
J = CHUNK // IDXW


@functools.lru_cache(maxsize=None)
def _build(B: int):
    info = plsc.get_sparse_core_info()
    nc, ns = info.num_cores, info.num_subcores
    nw = nc * ns
    assert B % (nw * CHUNK) == 0
    b_per_w = B // nw
    n_chunks = b_per_w // CHUNK
    rows_per_w = b_per_w // IDXW

    mesh = plsc.VectorSubcoreMesh(core_axis_name="c", subcore_axis_name="s")

    @functools.partial(
        pl.kernel,
        mesh=mesh,
        out_type=jax.ShapeDtypeStruct((B, DIM), jnp.float32),
        scratch_types=[
            pltpu.VMEM((J, IDXW), jnp.int32),
            pltpu.VMEM((CHUNK, DIM), jnp.float32),
            pltpu.SemaphoreType.DMA,
        ],
    )
    def emb(idx_hbm, table_hbm, out_hbm, idx_v, rows_v, sem):
        wid = lax.axis_index("s") * nc + lax.axis_index("c")
        row_base = wid * rows_per_w
        out_base = wid * b_per_w

        def body(g, carry):
            pltpu.sync_copy(idx_hbm.at[pl.ds(row_base + g * J, J)], idx_v)
            copies = [
                pltpu.async_copy(
                    table_hbm.at[idx_v.at[j]],
                    rows_v.at[pl.ds(j * IDXW, IDXW)],
                    sem,
                )
                for j in range(J)
            ]
            for c in copies:
                c.wait()
            pltpu.sync_copy(
                rows_v, out_hbm.at[pl.ds(out_base + g * CHUNK, CHUNK)]
            )
            return carry

        lax.fori_loop(0, n_chunks, body, 0)

    return emb


@jax.jit
def kernel(token_ids, weight):
    B = token_ids.size
    idx = token_ids.reshape(B // IDXW, IDXW).astype(jnp.int32)
    out = _build(B)(idx, weight)
    return out.reshape(token_ids.shape + (DIM,))


# speedup vs baseline: 1.7957x; 1.7957x over previous
import functools

import jax
import jax.numpy as jnp
from jax import lax
from jax.experimental import pallas as pl
from jax.experimental.pallas import tpu as pltpu
from jax.experimental.pallas import tpu_sc as plsc

DIM = 64
IDXW = 128
CHUNK = 512
J = CHUNK // IDXW


@functools.lru_cache(maxsize=None)
def _build(B: int):
    info = plsc.get_sparse_core_info()
    nc, ns = info.num_cores, info.num_subcores
    nw = nc * ns
    assert B % (nw * CHUNK) == 0
    b_per_w = B // nw
    n_chunks = b_per_w // CHUNK
    rows_per_w = b_per_w // IDXW

    mesh = plsc.VectorSubcoreMesh(core_axis_name="c", subcore_axis_name="s")

    @functools.partial(
        pl.kernel,
        mesh=mesh,
        out_type=jax.ShapeDtypeStruct((B, DIM), jnp.float32),
        scratch_types=[
            pltpu.VMEM((J, IDXW), jnp.int32),
            pltpu.VMEM((CHUNK, DIM), jnp.float32),
            pltpu.SemaphoreType.DMA,
        ],
        compiler_params=pltpu.CompilerParams(use_tc_tiling_on_sc=False),
    )
    def emb(idx_hbm, table_hbm, out_hbm, idx_v, rows_v, sem):
        wid = lax.axis_index("s") * nc + lax.axis_index("c")
        row_base = wid * rows_per_w
        out_base = wid * b_per_w

        def body(g, carry):
            pltpu.sync_copy(idx_hbm.at[pl.ds(row_base + g * J, J)], idx_v)
            copies = [
                pltpu.async_copy(
                    table_hbm.at[idx_v.at[j]],
                    rows_v.at[pl.ds(j * IDXW, IDXW)],
                    sem,
                )
                for j in range(J)
            ]
            for c in copies:
                c.wait()
            pltpu.sync_copy(
                rows_v, out_hbm.at[pl.ds(out_base + g * CHUNK, CHUNK)]
            )
            return carry

        lax.fori_loop(0, n_chunks, body, 0)

    return emb


@jax.jit
def kernel(token_ids, weight):
    B = token_ids.size
    idx = token_ids.reshape(B // IDXW, IDXW).astype(jnp.int32)
    out = _build(B)(idx, weight)
    return out.reshape(token_ids.shape + (DIM,))


# speedup vs baseline: 1.8760x; 1.0447x over previous
import functools

import jax
import jax.numpy as jnp
from jax import lax
from jax.experimental import pallas as pl
from jax.experimental.pallas import tpu as pltpu
from jax.experimental.pallas import tpu_sc as plsc

DIM = 64
IDXW = 128
CHUNK = 640
J = CHUNK // IDXW


@functools.lru_cache(maxsize=None)
def _build(B: int):
    info = plsc.get_sparse_core_info()
    nc, ns = info.num_cores, info.num_subcores
    nw = nc * ns
    assert B % (nw * 2 * CHUNK) == 0
    b_per_w = B // nw
    n_chunks = b_per_w // CHUNK
    rows_per_w = b_per_w // IDXW

    mesh = plsc.VectorSubcoreMesh(core_axis_name="c", subcore_axis_name="s")

    @functools.partial(
        pl.kernel,
        mesh=mesh,
        out_type=jax.ShapeDtypeStruct((B, DIM), jnp.float32),
        scratch_types=[
            pltpu.VMEM((rows_per_w, IDXW), jnp.int32),
            pltpu.VMEM((CHUNK, DIM), jnp.float32),
            pltpu.VMEM((CHUNK, DIM), jnp.float32),
            pltpu.SemaphoreType.DMA,
            pltpu.SemaphoreType.DMA,
            pltpu.SemaphoreType.DMA,
            pltpu.SemaphoreType.DMA,
        ],
        compiler_params=pltpu.CompilerParams(use_tc_tiling_on_sc=False),
    )
    def emb(idx_hbm, table_hbm, out_hbm, idx_all, rows0, rows1,
            sg0, sg1, so0, so1):
        wid = lax.axis_index("s") * nc + lax.axis_index("c")
        row_base = wid * rows_per_w
        out_base = wid * b_per_w
        rows = (rows0, rows1)
        sg = (sg0, sg1)
        so = (so0, so1)

        pltpu.sync_copy(idx_hbm.at[pl.ds(row_base, rows_per_w)], idx_all)

        def gather_start(c, b):
            for j in range(J):
                pltpu.async_copy(
                    table_hbm.at[idx_all.at[c * J + j]],
                    rows[b].at[pl.ds(j * IDXW, IDXW)],
                    sg[b],
                )

        def gather_wait(b):
            pltpu.make_async_copy(
                table_hbm.at[pl.ds(0, CHUNK)], rows[b], sg[b]
            ).wait()

        def out_start(c, b):
            pltpu.async_copy(
                rows[b], out_hbm.at[pl.ds(out_base + c * CHUNK, CHUNK)], so[b]
            )

        def out_drain(b):
            pltpu.make_async_copy(
                rows[b], out_hbm.at[pl.ds(out_base, CHUNK)], so[b]
            ).wait()

        gather_start(0, 0)

        def body(s, carry):
            for b in (0, 1):
                c = 2 * s + b
                nb = 1 - b

                @pl.when(c + 1 < n_chunks)
                def _prefetch():
                    @pl.when(c >= 1)
                    def _reuse():
                        out_drain(nb)

                    gather_start(c + 1, nb)

                gather_wait(b)
                out_start(c, b)
            return carry

        lax.fori_loop(0, n_chunks // 2, body, 0)
        out_drain(0)
        out_drain(1)

    return emb


@jax.jit
def kernel(token_ids, weight):
    B = token_ids.size
    idx = token_ids.reshape(B // IDXW, IDXW).astype(jnp.int32)
    out = _build(B)(idx, weight)
    return out.reshape(token_ids.shape + (DIM,))
